# baseline (speedup 1.0000x reference)
import jax
import jax.numpy as jnp
from jax import lax
from jax.experimental import pallas as pl
from jax.experimental.pallas import tpu as pltpu
from jax.experimental.pallas import tpu_sc as plsc

_B, _L, _V, _D = 4096, 50, 100000, 300
_NC, _NS = 2, 16
_NW = _NC * _NS
_LANES = 16
_ROWS = _B * _L
_RPW = _ROWS // _NW
_UPW = _RPW // _L
_CH = 128
_NCHUNK = _RPW // _CH
_MASK_ITERS = _RPW // _LANES

_DIV_MAGIC, _DIV_SHIFT = 5243, 18


def _sc_body(ids_hbm, len_hbm, table_hbm, emb_hbm, mask_hbm,
             idx_v, len_v, mask_v, buf, sem):
    wid = lax.axis_index("s") * _NC + lax.axis_index("c")
    row0 = wid * _RPW

    pltpu.sync_copy(len_hbm.at[pl.ds(wid * _UPW, _UPW)], len_v)

    def mask_body(j, carry):
        r = j * _LANES + lax.iota(jnp.int32, _LANES)
        u = (r * _DIV_MAGIC) >> _DIV_SHIFT
        pos = r - u * _L
        lens = plsc.load_gather(len_v, [u])
        mask_v[pl.ds(j * _LANES, _LANES)] = (pos < lens).astype(jnp.float32)
        return carry

    lax.fori_loop(0, _MASK_ITERS, mask_body, 0)
    pltpu.sync_copy(mask_v, mask_hbm.at[pl.ds(row0, _RPW)])

    def chunk_body(c, carry):
        cb = c * _CH
        pltpu.sync_copy(ids_hbm.at[pl.ds(row0 + cb, _CH)], idx_v)
        pltpu.async_copy(table_hbm.at[idx_v], buf, sem).wait()

        def row_body(r, rcarry):
            m = plsc.load_gather(mask_v, [jnp.full((_LANES,), cb + r, jnp.int32)])
            for k in range(_D // _LANES):
                buf[r, pl.ds(k * _LANES, _LANES)] *= m
            if _D % _LANES:
                buf[r, pl.ds(_D - _LANES, _LANES)] *= m
            return rcarry

        lax.fori_loop(0, _CH, row_body, 0)
        pltpu.sync_copy(buf, emb_hbm.at[pl.ds(row0 + cb, _CH)])
        return carry

    lax.fori_loop(0, _NCHUNK, chunk_body, 0)


@jax.jit
def _sc_call(ids, lens, table):
    mesh = plsc.VectorSubcoreMesh(
        core_axis_name="c", subcore_axis_name="s",
        num_cores=_NC, num_subcores=_NS)
    fn = pl.kernel(
        _sc_body,
        out_type=[
            jax.ShapeDtypeStruct((_ROWS, _D), jnp.float32),
            jax.ShapeDtypeStruct((_ROWS,), jnp.float32),
        ],
        mesh=mesh,
        scratch_types=[
            pltpu.VMEM((_CH,), jnp.int32),
            pltpu.VMEM((_UPW,), jnp.int32),
            pltpu.VMEM((_RPW,), jnp.float32),
            pltpu.VMEM((_CH, _D), jnp.float32),
            pltpu.SemaphoreType.DMA,
        ],
        compiler_params=pltpu.CompilerParams(
            needs_layout_passes=False, use_tc_tiling_on_sc=False),
    )
    return fn(ids, lens, table)


def kernel(token_ids, lengths, fasttext_table):
    assert token_ids.shape == (_B, _L) and fasttext_table.shape == (_V, _D)
    ids = token_ids.reshape(-1).astype(jnp.int32)
    lens = lengths.astype(jnp.int32)
    emb, mask = _sc_call(ids, lens, fasttext_table.astype(jnp.float32))
    return emb.reshape(_B, _L, _D), mask.reshape(_B, _L)


# speedup vs baseline: 1.3269x; 1.3269x over previous
import jax
import jax.numpy as jnp
from jax import lax
from jax.experimental import pallas as pl
from jax.experimental.pallas import tpu as pltpu
from jax.experimental.pallas import tpu_sc as plsc

_B, _L, _V, _D = 4096, 50, 100000, 300
_DP = 384
_NC, _NS = 2, 16
_NW = _NC * _NS
_LANES = 16
_ROWS = _B * _L
_RPW = _ROWS // _NW
_UPW = _RPW // _L
_CH = 128
_NCHUNK = _RPW // _CH
_MASK_ITERS = _RPW // _LANES
_MUL_SLICES = -(-_D // _LANES)

_DIV_MAGIC, _DIV_SHIFT = 5243, 18


def _sc_body(ids_hbm, len_hbm, table_hbm, emb_hbm, mask_hbm,
             idx_v, len_v, mask_v, buf, sem):
    wid = lax.axis_index("s") * _NC + lax.axis_index("c")
    row0 = wid * _RPW

    pltpu.sync_copy(len_hbm.at[pl.ds(wid * _UPW, _UPW)], len_v)

    def mask_body(j, carry):
        r = j * _LANES + lax.iota(jnp.int32, _LANES)
        u = (r * _DIV_MAGIC) >> _DIV_SHIFT
        pos = r - u * _L
        lens = plsc.load_gather(len_v, [u])
        mask_v[pl.ds(j * _LANES, _LANES)] = (pos < lens).astype(jnp.float32)
        return carry

    lax.fori_loop(0, _MASK_ITERS, mask_body, 0)
    pltpu.sync_copy(mask_v, mask_hbm.at[pl.ds(row0, _RPW)])

    def chunk_body(c, carry):
        cb = c * _CH
        pltpu.sync_copy(ids_hbm.at[pl.ds(row0 + cb, _CH)], idx_v)
        pltpu.async_copy(table_hbm.at[idx_v], buf, sem).wait()

        def row_body(r, rcarry):
            m = plsc.load_gather(mask_v, [jnp.full((_LANES,), cb + r, jnp.int32)])
            for k in range(_MUL_SLICES):
                buf[r, pl.ds(k * _LANES, _LANES)] *= m
            return rcarry

        lax.fori_loop(0, _CH, row_body, 0)
        pltpu.sync_copy(buf, emb_hbm.at[pl.ds(row0 + cb, _CH)])
        return carry

    lax.fori_loop(0, _NCHUNK, chunk_body, 0)


@jax.jit
def _sc_call(ids, lens, table_pad):
    mesh = plsc.VectorSubcoreMesh(
        core_axis_name="c", subcore_axis_name="s",
        num_cores=_NC, num_subcores=_NS)
    fn = pl.kernel(
        _sc_body,
        out_type=[
            jax.ShapeDtypeStruct((_ROWS, _DP), jnp.float32),
            jax.ShapeDtypeStruct((_ROWS,), jnp.float32),
        ],
        mesh=mesh,
        scratch_types=[
            pltpu.VMEM((_CH,), jnp.int32),
            pltpu.VMEM((_UPW,), jnp.int32),
            pltpu.VMEM((_RPW,), jnp.float32),
            pltpu.VMEM((_CH, _DP), jnp.float32),
            pltpu.SemaphoreType.DMA,
        ],
        compiler_params=pltpu.CompilerParams(
            needs_layout_passes=False, use_tc_tiling_on_sc=True),
    )
    return fn(ids, lens, table_pad)


def kernel(token_ids, lengths, fasttext_table):
    assert token_ids.shape == (_B, _L) and fasttext_table.shape == (_V, _D)
    ids = token_ids.reshape(-1).astype(jnp.int32)
    lens = lengths.astype(jnp.int32)
    table_pad = jnp.pad(fasttext_table.astype(jnp.float32),
                        ((0, 0), (0, _DP - _D)))
    emb_pad, mask = _sc_call(ids, lens, table_pad)
    return emb_pad[:, :_D].reshape(_B, _L, _D), mask.reshape(_B, _L)
